# Initial kernel scaffold; baseline (speedup 1.0000x reference)
import jax
import jax.numpy as jnp
from jax.experimental import pallas as pl


def kernel(x, edge_index, W1, b1, W2, b2):
    raise NotImplementedError("write your pallas kernel here")


# speedup vs baseline: 11.3689x; 11.3689x over previous
import functools

import jax
import jax.numpy as jnp
from jax import lax
from jax.experimental import pallas as pl
from jax.experimental.pallas import tpu as pltpu
from jax.experimental.pallas import tpu_sc as plsc

N_NODES = 10000
IN_CH = 128
HID_CH = 128
OUT_CH = 64
N_EDGES = 320000

NC, NS, LANES = 2, 16, 16
NW = NC * NS

EPT_RAW = N_EDGES // NW
CHUNK = 128
NCHUNK = -(-N_EDGES // (NW * CHUNK))
EPT = NCHUNK * CHUNK
HIST = NCHUNK * CHUNK
NPAD = 10112
RPT = NPAD // NS

_mesh = plsc.VectorSubcoreMesh(core_axis_name="c", subcore_axis_name="s")


def _deg_body(dst_hbm, out_hbm, idx_v, hist_v):
    c = lax.axis_index("c")
    s = lax.axis_index("s")
    wid = c * NS + s
    pltpu.sync_copy(dst_hbm.at[wid], idx_v)
    zero16 = jnp.zeros((LANES,), jnp.float32)

    def zrow(r, carry):
        for cc in range(CHUNK // LANES):
            hist_v[r, pl.ds(cc * LANES, LANES)] = zero16
        return carry

    lax.fori_loop(0, NCHUNK, zrow, 0)

    ones = jnp.ones((LANES,), jnp.float32)

    def edge(j, carry):
        idx = idx_v[pl.ds(j * LANES, LANES)]
        hi = lax.shift_right_logical(idx, 7)
        lo = lax.bitwise_and(idx, 127)
        plsc.addupdate_scatter(hist_v, [hi, lo], ones)
        return carry

    lax.fori_loop(0, EPT_RAW // LANES, edge, 0)
    pltpu.sync_copy(hist_v, out_hbm.at[wid])


_deg_call = pl.kernel(
    _deg_body,
    out_type=jax.ShapeDtypeStruct((NW, NCHUNK, CHUNK), jnp.float32),
    mesh=_mesh,
    scratch_types=[
        pltpu.VMEM((EPT_RAW,), jnp.int32),
        pltpu.VMEM((NCHUNK, CHUNK), jnp.float32),
    ],
    compiler_params=pltpu.CompilerParams(needs_layout_passes=False),
)


def _make_agg(d):
    def body(hs_hbm, src_hbm, dst_hbm, zeros_hbm, out_hbm,
             src_v, dst_v, rows_v, acc_sh, sem_g, sem_s):
        c = lax.axis_index("c")
        s = lax.axis_index("s")
        wid = c * NS + s
        pltpu.sync_copy(src_hbm.at[wid], src_v)
        pltpu.sync_copy(dst_hbm.at[wid], dst_v)
        pltpu.sync_copy(zeros_hbm.at[pl.ds(s * RPT, RPT)],
                        acc_sh.at[pl.ds(s * RPT, RPT)])
        plsc.subcore_barrier()

        def chunk(j, carry):
            pltpu.async_copy(hs_hbm.at[src_v.at[j]], rows_v, sem_g).wait()
            pltpu.async_copy(rows_v, acc_sh.at[dst_v.at[j]], sem_s, add=True).wait()
            return carry

        lax.fori_loop(0, NCHUNK, chunk, 0)
        plsc.subcore_barrier()
        pltpu.sync_copy(acc_sh.at[pl.ds(s * RPT, RPT)],
                        out_hbm.at[c].at[pl.ds(s * RPT, RPT)])

    return pl.kernel(
        body,
        out_type=jax.ShapeDtypeStruct((NC, NPAD, d), jnp.float32),
        mesh=_mesh,
        scratch_types=[
            pltpu.VMEM((NCHUNK, CHUNK), jnp.int32),
            pltpu.VMEM((NCHUNK, CHUNK), jnp.int32),
            pltpu.VMEM((CHUNK, d), jnp.float32),
            pltpu.VMEM_SHARED((NPAD, d), jnp.float32),
            pltpu.SemaphoreType.DMA,
            pltpu.SemaphoreType.DMA,
        ],
        compiler_params=pltpu.CompilerParams(use_tc_tiling_on_sc=False),
    )


_agg_hid = _make_agg(HID_CH)
_agg_out = _make_agg(OUT_CH)


def _dinv_body(degp_ref, out_ref):
    deg = jnp.sum(degp_ref[...], axis=0) + 1.0
    out_ref[...] = lax.rsqrt(deg)


def _hs1_body(x_ref, w_ref, dinv_ref, out_ref):
    out_ref[...] = (
        jnp.dot(x_ref[...], w_ref[...], preferred_element_type=jnp.float32)
        * dinv_ref[...]
    )


def _hs2_body(a0_ref, a1_ref, hs1_ref, dinv_ref, b1_ref, w2_ref, out_ref):
    h = (a0_ref[...] + a1_ref[...] + hs1_ref[...]) * dinv_ref[...] + b1_ref[...]
    h = jnp.maximum(h, 0.0)
    out_ref[...] = (
        jnp.dot(h, w2_ref[...], preferred_element_type=jnp.float32)
        * dinv_ref[...]
    )


def _z_body(a0_ref, a1_ref, hs2_ref, dinv_ref, b2_ref, out_ref):
    out_ref[...] = (
        (a0_ref[...] + a1_ref[...] + hs2_ref[...]) * dinv_ref[...] + b2_ref[...]
    )


def _adj_body(zr_ref, zc_ref, out_ref):
    g = lax.dot_general(zr_ref[...], zc_ref[...],
                        (((1,), (1,)), ((), ())),
                        preferred_element_type=jnp.float32)
    out_ref[...] = jax.nn.sigmoid(g)


_BM = 512
_GM = -(-N_NODES // _BM)

_dinv_call = pl.pallas_call(
    _dinv_body,
    out_shape=jax.ShapeDtypeStruct((NCHUNK, CHUNK), jnp.float32),
    in_specs=[pl.BlockSpec((NW, NCHUNK, CHUNK), lambda: (0, 0, 0))],
    out_specs=pl.BlockSpec((NCHUNK, CHUNK), lambda: (0, 0)),
)

_hs1_call = pl.pallas_call(
    _hs1_body,
    grid=(_GM,),
    out_shape=jax.ShapeDtypeStruct((N_NODES, HID_CH), jnp.float32),
    in_specs=[
        pl.BlockSpec((_BM, IN_CH), lambda i: (i, 0)),
        pl.BlockSpec((IN_CH, HID_CH), lambda i: (0, 0)),
        pl.BlockSpec((_BM, 1), lambda i: (i, 0)),
    ],
    out_specs=pl.BlockSpec((_BM, HID_CH), lambda i: (i, 0)),
)

_hs2_call = pl.pallas_call(
    _hs2_body,
    grid=(_GM,),
    out_shape=jax.ShapeDtypeStruct((N_NODES, OUT_CH), jnp.float32),
    in_specs=[
        pl.BlockSpec((_BM, HID_CH), lambda i: (i, 0)),
        pl.BlockSpec((_BM, HID_CH), lambda i: (i, 0)),
        pl.BlockSpec((_BM, HID_CH), lambda i: (i, 0)),
        pl.BlockSpec((_BM, 1), lambda i: (i, 0)),
        pl.BlockSpec((1, HID_CH), lambda i: (0, 0)),
        pl.BlockSpec((HID_CH, OUT_CH), lambda i: (0, 0)),
    ],
    out_specs=pl.BlockSpec((_BM, OUT_CH), lambda i: (i, 0)),
)

_z_call = pl.pallas_call(
    _z_body,
    grid=(_GM,),
    out_shape=jax.ShapeDtypeStruct((N_NODES, OUT_CH), jnp.float32),
    in_specs=[
        pl.BlockSpec((_BM, OUT_CH), lambda i: (i, 0)),
        pl.BlockSpec((_BM, OUT_CH), lambda i: (i, 0)),
        pl.BlockSpec((_BM, OUT_CH), lambda i: (i, 0)),
        pl.BlockSpec((_BM, 1), lambda i: (i, 0)),
        pl.BlockSpec((1, OUT_CH), lambda i: (0, 0)),
    ],
    out_specs=pl.BlockSpec((_BM, OUT_CH), lambda i: (i, 0)),
)

_BN = 2048
_GN = -(-N_NODES // _BN)

_adj_call = pl.pallas_call(
    _adj_body,
    grid=(_GM, _GN),
    out_shape=jax.ShapeDtypeStruct((N_NODES, N_NODES), jnp.float32),
    in_specs=[
        pl.BlockSpec((_BM, OUT_CH), lambda i, j: (i, 0)),
        pl.BlockSpec((_BN, OUT_CH), lambda i, j: (j, 0)),
    ],
    out_specs=pl.BlockSpec((_BM, _BN), lambda i, j: (i, j)),
    compiler_params=pltpu.CompilerParams(
        dimension_semantics=("parallel", "parallel")),
)


def kernel(x, edge_index, W1, b1, W2, b2):
    ei = edge_index.astype(jnp.int32)
    src, dst = ei[0], ei[1]

    degp = _deg_call(dst.reshape(NW, EPT_RAW))
    dinv2d = _dinv_call(degp)
    dinv = dinv2d.reshape(-1)[:N_NODES, None]

    pad = NW * EPT - N_EDGES
    srcp = jnp.concatenate([src, jnp.zeros((pad,), jnp.int32)]).reshape(
        NW, NCHUNK, CHUNK)
    dstp = jnp.concatenate([dst, jnp.full((pad,), N_NODES, jnp.int32)]).reshape(
        NW, NCHUNK, CHUNK)

    hs1 = _hs1_call(x, W1, dinv)
    aggp1 = _agg_hid(hs1, srcp, dstp, jnp.zeros((NPAD, HID_CH), jnp.float32))
    hs2 = _hs2_call(aggp1[0, :N_NODES], aggp1[1, :N_NODES], hs1, dinv,
                    b1.reshape(1, HID_CH), W2)
    aggp2 = _agg_out(hs2, srcp, dstp, jnp.zeros((NPAD, OUT_CH), jnp.float32))
    z = _z_call(aggp2[0, :N_NODES], aggp2[1, :N_NODES], hs2, dinv,
                b2.reshape(1, OUT_CH))
    adj = _adj_call(z, z)
    return (z, adj)
